# baseline (speedup 1.0000x reference)
import functools

import jax
import jax.numpy as jnp
from jax import lax
from jax.experimental import pallas as pl
from jax.experimental.pallas import tpu as pltpu
from jax.experimental.pallas import tpu_sc as plsc

_N = 100000
_D = 128
_K = 1024
_BLK = 2048
_NBLK = 49
_NPAD = _NBLK * _BLK
_NW = 16
_CHUNK = _NPAD // _NW
_CVREG = _CHUNK // 16
_OUTPAD = _K + 16
_SELCAP = 1152
_NBATCH = _SELCAP // 128


def _keys_body(x_ref, w_ref, out_ref):
    i = pl.program_id(0)
    xb = x_ref[...]
    w = w_ref[...]
    norm = jnp.sqrt(jnp.sum(w * w))
    y = (
        lax.dot_general(
            xb, w, (((1,), (0,)), ((), ())), preferred_element_type=jnp.float32
        )
        / norm
    ).reshape(1, _BLK)
    gid = i * _BLK + lax.broadcasted_iota(jnp.int32, (1, _BLK), 1)
    valid = gid < _N
    yb = jnp.where(valid, y, -jnp.inf)
    u = lax.bitcast_convert_type(yb, jnp.uint32)
    ku = jnp.where((u & jnp.uint32(0x80000000)) != 0, ~u, u | jnp.uint32(0x80000000))
    ki = lax.bitcast_convert_type(ku ^ jnp.uint32(0x80000000), jnp.int32)
    out_ref[...] = ki.reshape(_BLK)


def _tc_keys(x, w):
    return pl.pallas_call(
        _keys_body,
        grid=(_NBLK,),
        in_specs=[
            pl.BlockSpec((_BLK, _D), lambda i: (i, 0)),
            pl.BlockSpec((_D, 1), lambda i: (0, 0)),
        ],
        out_specs=pl.BlockSpec((_BLK,), lambda i: (i,)),
        out_shape=jax.ShapeDtypeStruct((_NPAD,), jnp.int32),
    )(x, w)


def _iota16():
    return lax.broadcasted_iota(jnp.int32, (16,), 0)


def _bin_of(kv, shift, level0):
    b = lax.shift_right_arithmetic(kv, shift) & 255
    if level0:
        b = b ^ 128
    return b


def _sc_body(keys_hbm, x_hbm, outk_hbm, outi_hbm, outx_hbm,
             k_v, hist2, histv, ghl, cand_k, cand_i, sel_k, sel_i,
             cvb, cnt2, idxg, oidx, gbuf, shist, scnt, sem):
    core = lax.axis_index("c")
    wid = lax.axis_index("s")
    i16 = _iota16()

    @pl.when(core == 0)
    def _main():
        base = wid * _CHUNK
        pltpu.sync_copy(keys_hbm.at[pl.ds(base, _CHUNK)], k_v)

        def zero_hist():
            def zrow(rt, _):
                hist2[pl.ds(rt * 16, 16)] = jnp.zeros((16,), jnp.int32)
                return 0
            lax.fori_loop(0, 256, zrow, 0)

        def build_hist(n_scalar, from_cand, level, shift):
            zero_hist()
            ones = jnp.ones((16,), jnp.int32)
            nv = (n_scalar + 15) // 16

            def hb(i, _):
                if from_cand:
                    kv = cand_k[pl.ds(i * 16, 16)]
                else:
                    kv = k_v[pl.ds(i * 16, 16)]
                m = (i * 16 + i16) < n_scalar
                b = _bin_of(kv, shift, level == 0)
                slot = i16 * 256 + b
                cur = plsc.load_gather(hist2, [slot])
                plsc.store_scatter(hist2, [slot], cur + jnp.where(m, ones, 0))
                return 0

            lax.fori_loop(0, nv, hb, 0)

            def cb(t, _):
                acc = jnp.zeros((16,), jnp.int32)
                for r in range(16):
                    acc = acc + hist2[pl.ds(r * 256 + t * 16, 16)]
                histv[pl.ds(t * 16, 16)] = acc
                return 0

            lax.fori_loop(0, 16, cb, 0)

        def global_hist_and_bin(target_rem):
            pltpu.sync_copy(histv, shist.at[pl.ds(wid * 256, 256)])
            plsc.subcore_barrier()
            pltpu.sync_copy(shist, ghl)

            def scan_step(vi, carry):
                b_star, above, run, done = carry
                v = 15 - vi
                g = jnp.zeros((16,), jnp.int32)
                for w in range(_NW):
                    g = g + ghl[pl.ds(w * 256 + v * 16, 16)]
                rev = lax.rev(g, (0,))
                cs = plsc.cumsum(rev)
                tot = jnp.max(cs)
                hit = (run + cs) >= target_rem
                nhit = jnp.sum(hit.astype(jnp.int32))
                jstar = jnp.min(jnp.where(hit, i16, 16))
                csj = jnp.sum(jnp.where(i16 == jstar, cs, 0))
                gj = jnp.sum(jnp.where(i16 == jstar, rev, 0))
                found = jnp.logical_and(jnp.logical_not(done), nhit > 0)
                b_star = jnp.where(found, v * 16 + 15 - jstar, b_star)
                above = jnp.where(found, run + csj - gj, above)
                run = jnp.where(done | found, run, run + tot)
                done = done | found
                return b_star, above, run, done

            b_star, above, _, _ = lax.fori_loop(
                0, 16, scan_step,
                (jnp.int32(0), jnp.int32(0), jnp.int32(0), jnp.bool_(False)),
            )
            return b_star, above

        def compact(n_scalar, from_kv, level, shift, b_star, sel_off0):
            nv = (n_scalar + 15) // 16

            def cp(i, carry):
                soff, coff = carry
                if from_kv:
                    kv = k_v[pl.ds(i * 16, 16)]
                    iv = base + i * 16 + i16
                else:
                    kv = cand_k[pl.ds(i * 16, 16)]
                    iv = cand_i[pl.ds(i * 16, 16)]
                valid = (i * 16 + i16) < n_scalar
                b = _bin_of(kv, shift, level == 0)
                m_gt = jnp.logical_and(b > b_star, valid)
                m_eq = jnp.logical_and(b == b_star, valid)
                plsc.store_compressed(sel_k.at[pl.ds(soff, 16)], kv, mask=m_gt)
                plsc.store_compressed(sel_i.at[pl.ds(soff, 16)], iv, mask=m_gt)
                plsc.store_compressed(cand_k.at[pl.ds(coff, 16)], kv, mask=m_eq)
                plsc.store_compressed(cand_i.at[pl.ds(coff, 16)], iv, mask=m_eq)
                soff = soff + jnp.sum(m_gt.astype(jnp.int32))
                coff = coff + jnp.sum(m_eq.astype(jnp.int32))
                return soff, coff

            return lax.fori_loop(0, nv, cp, (sel_off0, jnp.int32(0)))

        n_cur = jnp.int32(_CHUNK)
        sel_off = jnp.int32(0)
        cnt_gt = jnp.int32(0)
        for level, shift in enumerate((24, 16, 8, 0)):
            from_kv = level == 0
            build_hist(n_cur, not from_kv, level, shift)
            b_star, above = global_hist_and_bin(_K - cnt_gt)
            sel_off, n_cur = compact(n_cur, from_kv, level, shift, b_star, sel_off)
            cnt_gt = cnt_gt + above
            plsc.subcore_barrier()

        g_w = sel_off
        e_w = n_cur

        cvec = jnp.where(i16 == 0, g_w, jnp.where(i16 == 1, e_w, 0))
        cvb[pl.ds(0, 16)] = cvec
        pltpu.sync_copy(cvb, scnt.at[pl.ds(wid * 16, 16)])
        plsc.subcore_barrier()
        pltpu.sync_copy(scnt, cnt2)
        g_all = plsc.load_gather(cnt2, [i16 * 16])
        e_all = plsc.load_gather(cnt2, [i16 * 16 + 1])
        g_cum = plsc.cumsum(g_all)
        e_cum = plsc.cumsum(e_all)
        off_g = jnp.sum(jnp.where(i16 == wid, g_cum - g_all, 0))
        off_e = jnp.sum(jnp.where(i16 == wid, e_cum - e_all, 0))
        total_gt = jnp.sum(g_all)
        r_ties = _K - total_gt
        keep_w = jnp.clip(r_ties - off_e, 0, e_w)

        def ap(i, _):
            kv = cand_k[pl.ds(i * 16, 16)]
            iv = cand_i[pl.ds(i * 16, 16)]
            m = (i * 16 + i16) < keep_w
            plsc.store_compressed(sel_k.at[pl.ds(g_w + i * 16, 16)], kv, mask=m)
            plsc.store_compressed(sel_i.at[pl.ds(g_w + i * 16, 16)], iv, mask=m)
            return 0

        lax.fori_loop(0, (keep_w + 15) // 16, ap, 0)
        n_tot = g_w + keep_w

        def bi(rt, _):
            j = rt // 8
            t = rt % 8
            p = j * 128 + t * 16 + i16
            valid = p < n_tot
            iv = sel_i[pl.ds(j * 128 + t * 16, 16)]
            idxg[j, pl.ds(t * 16, 16)] = jnp.where(valid, iv, 0)
            slot = jnp.where(
                p < g_w,
                off_g + p,
                jnp.where(valid, total_gt + off_e + (p - g_w), _K + wid),
            )
            oidx[j, pl.ds(t * 16, 16)] = slot
            return 0

        lax.fori_loop(0, _NBATCH * 8, bi, 0)

        for j in range(_NBATCH):
            @pl.when(j * 128 < n_tot)
            def _s(j=j):
                pltpu.async_copy(x_hbm.at[idxg.at[j]], gbuf, sem).wait()
                pltpu.async_copy(gbuf, outx_hbm.at[oidx.at[j]], sem).wait()
                pltpu.async_copy(sel_k.at[pl.ds(j * 128, 128)], outk_hbm.at[oidx.at[j]], sem).wait()
                pltpu.async_copy(sel_i.at[pl.ds(j * 128, 128)], outi_hbm.at[oidx.at[j]], sem).wait()


def _sc_select(keys, x):
    mesh = plsc.VectorSubcoreMesh(core_axis_name="c", subcore_axis_name="s")
    f = functools.partial(
        pl.kernel,
        mesh=mesh,
        out_type=[
            jax.ShapeDtypeStruct((_OUTPAD,), jnp.int32),
            jax.ShapeDtypeStruct((_OUTPAD,), jnp.int32),
            jax.ShapeDtypeStruct((_OUTPAD, _D), jnp.float32),
        ],
        scratch_types=[
            pltpu.VMEM((_CHUNK,), jnp.int32),
            pltpu.VMEM((16 * 256,), jnp.int32),
            pltpu.VMEM((256,), jnp.int32),
            pltpu.VMEM((_NW * 256,), jnp.int32),
            pltpu.VMEM((_CHUNK + 16,), jnp.int32),
            pltpu.VMEM((_CHUNK + 16,), jnp.int32),
            pltpu.VMEM((_SELCAP + 16,), jnp.int32),
            pltpu.VMEM((_SELCAP + 16,), jnp.int32),
            pltpu.VMEM((16,), jnp.int32),
            pltpu.VMEM((_NW * 16,), jnp.int32),
            pltpu.VMEM((_NBATCH, 128), jnp.int32),
            pltpu.VMEM((_NBATCH, 128), jnp.int32),
            pltpu.VMEM((128, _D), jnp.float32),
            pltpu.VMEM_SHARED((_NW * 256,), jnp.int32),
            pltpu.VMEM_SHARED((_NW * 16,), jnp.int32),
            pltpu.SemaphoreType.DMA,
        ],
        compiler_params=pltpu.CompilerParams(
            needs_layout_passes=False, use_tc_tiling_on_sc=False
        ),
    )(_sc_body)
    return f(keys, x)


def _rank_body(k_ref, i_ref, x_ref, out_ref):
    keys = k_ref[...].reshape(1, _K)
    idx = i_ref[...].reshape(1, _K)
    x0 = x_ref[...][:, 0].reshape(1, _K)
    kcol = keys.reshape(_K, 1)
    icol = idx.reshape(_K, 1)
    gt = (keys > kcol).astype(jnp.int32)
    tie = jnp.logical_and(keys == kcol, idx < icol).astype(jnp.int32)
    rank = jnp.sum(gt + tie, axis=1).reshape(1, _K)
    ku = lax.bitcast_convert_type(keys, jnp.uint32) ^ jnp.uint32(0x80000000)
    u = jnp.where((ku & jnp.uint32(0x80000000)) != 0, ku ^ jnp.uint32(0x80000000), ~ku)
    y = lax.bitcast_convert_type(u, jnp.float32)
    prod = (x0 * jax.nn.sigmoid(y)).reshape(_K, 1)
    row = lax.broadcasted_iota(jnp.int32, (_K, _K), 0)
    p2 = (row == rank).astype(jnp.float32)
    out = lax.dot_general(
        p2, prod, (((1,), (0,)), ((), ())),
        preferred_element_type=jnp.float32,
        precision=lax.Precision.HIGHEST,
    )
    out_ref[...] = out


def _tc_rank(ck, ci, cx):
    return pl.pallas_call(
        _rank_body,
        out_shape=jax.ShapeDtypeStruct((_K, 1), jnp.float32),
    )(ck, ci, cx)


def kernel(x, learnable_vector):
    keys = _tc_keys(x, learnable_vector)
    ck, ci, cx = _sc_select(keys, x)
    return _tc_rank(ck[:_K], ci[:_K], cx[:_K])


# speedup vs baseline: 1.0818x; 1.0818x over previous
import functools

import jax
import jax.numpy as jnp
from jax import lax
from jax.experimental import pallas as pl
from jax.experimental.pallas import tpu as pltpu
from jax.experimental.pallas import tpu_sc as plsc

_N = 100000
_D = 128
_K = 1024
_BLK = 2048
_NBLK = 49
_NPAD = _NBLK * _BLK
_NW = 16
_CHUNK = _NPAD // _NW
_CVREG = _CHUNK // 16
_OUTPAD = _K + 16
_SELCAP = 1152
_NBATCH = _SELCAP // 128


def _keys_body(x_ref, w_ref, out_ref):
    i = pl.program_id(0)
    xb = x_ref[...]
    w = w_ref[...]
    norm = jnp.sqrt(jnp.sum(w * w))
    y = (
        lax.dot_general(
            xb, w, (((1,), (0,)), ((), ())), preferred_element_type=jnp.float32
        )
        / norm
    ).reshape(1, _BLK)
    gid = i * _BLK + lax.broadcasted_iota(jnp.int32, (1, _BLK), 1)
    valid = gid < _N
    yb = jnp.where(valid, y, -jnp.inf)
    u = lax.bitcast_convert_type(yb, jnp.uint32)
    ku = jnp.where((u & jnp.uint32(0x80000000)) != 0, ~u, u | jnp.uint32(0x80000000))
    ki = lax.bitcast_convert_type(ku ^ jnp.uint32(0x80000000), jnp.int32)
    out_ref[...] = ki.reshape(_BLK)


def _tc_keys(x, w):
    return pl.pallas_call(
        _keys_body,
        grid=(_NBLK,),
        in_specs=[
            pl.BlockSpec((_BLK, _D), lambda i: (i, 0)),
            pl.BlockSpec((_D, 1), lambda i: (0, 0)),
        ],
        out_specs=pl.BlockSpec((_BLK,), lambda i: (i,)),
        out_shape=jax.ShapeDtypeStruct((_NPAD,), jnp.int32),
    )(x, w)


def _iota16():
    return lax.broadcasted_iota(jnp.int32, (16,), 0)


def _bin_of(kv, shift, level0):
    b = lax.shift_right_arithmetic(kv, shift) & 255
    if level0:
        b = b ^ 128
    return b


def _sc_body(keys_hbm, xflat_hbm, outk_hbm, outi_hbm, outx_hbm,
             k_v, hist2, histv, ghl, cand_k, cand_i, sel_k, sel_i, selx,
             cvb, cnt2, idxg, oidx, shist, scnt, sem):
    core = lax.axis_index("c")
    wid = lax.axis_index("s")
    i16 = _iota16()

    @pl.when(core == 0)
    def _main():
        base = wid * _CHUNK
        pltpu.sync_copy(keys_hbm.at[pl.ds(base, _CHUNK)], k_v)

        def zero_hist():
            def zrow(rt, _):
                hist2[pl.ds(rt * 16, 16)] = jnp.zeros((16,), jnp.int32)
                return 0
            lax.fori_loop(0, 256, zrow, 0)

        def build_hist(n_scalar, from_cand, level, shift):
            zero_hist()
            ones = jnp.ones((16,), jnp.int32)
            nv = (n_scalar + 15) // 16

            def hb(i, _):
                if from_cand:
                    kv = cand_k[pl.ds(i * 16, 16)]
                else:
                    kv = k_v[pl.ds(i * 16, 16)]
                m = (i * 16 + i16) < n_scalar
                b = _bin_of(kv, shift, level == 0)
                slot = i16 * 256 + b
                cur = plsc.load_gather(hist2, [slot])
                plsc.store_scatter(hist2, [slot], cur + jnp.where(m, ones, 0))
                return 0

            lax.fori_loop(0, nv, hb, 0)

            def cb(t, _):
                acc = jnp.zeros((16,), jnp.int32)
                for r in range(16):
                    acc = acc + hist2[pl.ds(r * 256 + t * 16, 16)]
                histv[pl.ds(t * 16, 16)] = acc
                return 0

            lax.fori_loop(0, 16, cb, 0)

        def global_hist_and_bin(target_rem):
            pltpu.sync_copy(histv, shist.at[pl.ds(wid * 256, 256)])
            plsc.subcore_barrier()
            pltpu.sync_copy(shist, ghl)

            def scan_step(vi, carry):
                b_star, above, run, done = carry
                v = 15 - vi
                g = jnp.zeros((16,), jnp.int32)
                for w in range(_NW):
                    g = g + ghl[pl.ds(w * 256 + v * 16, 16)]
                rev = lax.rev(g, (0,))
                cs = plsc.cumsum(rev)
                tot = jnp.max(cs)
                hit = (run + cs) >= target_rem
                nhit = jnp.sum(hit.astype(jnp.int32))
                jstar = jnp.min(jnp.where(hit, i16, 16))
                csj = jnp.sum(jnp.where(i16 == jstar, cs, 0))
                gj = jnp.sum(jnp.where(i16 == jstar, rev, 0))
                found = jnp.logical_and(jnp.logical_not(done), nhit > 0)
                b_star = jnp.where(found, v * 16 + 15 - jstar, b_star)
                above = jnp.where(found, run + csj - gj, above)
                run = jnp.where(done | found, run, run + tot)
                done = done | found
                return b_star, above, run, done

            b_star, above, _, _ = lax.fori_loop(
                0, 16, scan_step,
                (jnp.int32(0), jnp.int32(0), jnp.int32(0), jnp.bool_(False)),
            )
            return b_star, above

        def compact(n_scalar, from_kv, level, shift, b_star, sel_off0):
            nv = (n_scalar + 15) // 16

            def cp(i, carry):
                soff, coff = carry
                if from_kv:
                    kv = k_v[pl.ds(i * 16, 16)]
                    iv = base + i * 16 + i16
                else:
                    kv = cand_k[pl.ds(i * 16, 16)]
                    iv = cand_i[pl.ds(i * 16, 16)]
                valid = (i * 16 + i16) < n_scalar
                b = _bin_of(kv, shift, level == 0)
                m_gt = jnp.logical_and(b > b_star, valid)
                m_eq = jnp.logical_and(b == b_star, valid)
                plsc.store_compressed(sel_k.at[pl.ds(soff, 16)], kv, mask=m_gt)
                plsc.store_compressed(sel_i.at[pl.ds(soff, 16)], iv, mask=m_gt)
                plsc.store_compressed(cand_k.at[pl.ds(coff, 16)], kv, mask=m_eq)
                plsc.store_compressed(cand_i.at[pl.ds(coff, 16)], iv, mask=m_eq)
                soff = soff + jnp.sum(m_gt.astype(jnp.int32))
                coff = coff + jnp.sum(m_eq.astype(jnp.int32))
                return soff, coff

            return lax.fori_loop(0, nv, cp, (sel_off0, jnp.int32(0)))

        n_cur = jnp.int32(_CHUNK)
        sel_off = jnp.int32(0)
        cnt_gt = jnp.int32(0)
        for level, shift in enumerate((24, 16, 8, 0)):
            from_kv = level == 0
            build_hist(n_cur, not from_kv, level, shift)
            b_star, above = global_hist_and_bin(_K - cnt_gt)
            sel_off, n_cur = compact(n_cur, from_kv, level, shift, b_star, sel_off)
            cnt_gt = cnt_gt + above
            plsc.subcore_barrier()

        g_w = sel_off
        e_w = n_cur

        cvec = jnp.where(i16 == 0, g_w, jnp.where(i16 == 1, e_w, 0))
        cvb[pl.ds(0, 16)] = cvec
        pltpu.sync_copy(cvb, scnt.at[pl.ds(wid * 16, 16)])
        plsc.subcore_barrier()
        pltpu.sync_copy(scnt, cnt2)
        g_all = plsc.load_gather(cnt2, [i16 * 16])
        e_all = plsc.load_gather(cnt2, [i16 * 16 + 1])
        g_cum = plsc.cumsum(g_all)
        e_cum = plsc.cumsum(e_all)
        off_g = jnp.sum(jnp.where(i16 == wid, g_cum - g_all, 0))
        off_e = jnp.sum(jnp.where(i16 == wid, e_cum - e_all, 0))
        total_gt = jnp.sum(g_all)
        r_ties = _K - total_gt
        keep_w = jnp.clip(r_ties - off_e, 0, e_w)

        def ap(i, _):
            kv = cand_k[pl.ds(i * 16, 16)]
            iv = cand_i[pl.ds(i * 16, 16)]
            m = (i * 16 + i16) < keep_w
            plsc.store_compressed(sel_k.at[pl.ds(g_w + i * 16, 16)], kv, mask=m)
            plsc.store_compressed(sel_i.at[pl.ds(g_w + i * 16, 16)], iv, mask=m)
            return 0

        lax.fori_loop(0, (keep_w + 15) // 16, ap, 0)
        n_tot = g_w + keep_w

        def bi(rt, _):
            j = rt // 8
            t = rt % 8
            p = j * 128 + t * 16 + i16
            valid = p < n_tot
            iv = sel_i[pl.ds(j * 128 + t * 16, 16)]
            idxg[j, pl.ds(t * 16, 16)] = jnp.where(valid, iv * 128, 0)
            slot = jnp.where(
                p < g_w,
                off_g + p,
                jnp.where(valid, total_gt + off_e + (p - g_w), _K + wid),
            )
            oidx[j, pl.ds(t * 16, 16)] = slot
            return 0

        lax.fori_loop(0, _NBATCH * 8, bi, 0)

        for j in range(_NBATCH):
            @pl.when(j * 128 < n_tot)
            def _s(j=j):
                pltpu.async_copy(
                    xflat_hbm.at[idxg.at[j]], selx.at[pl.ds(j * 128, 128)], sem
                ).wait()
                pltpu.async_copy(selx.at[pl.ds(j * 128, 128)], outx_hbm.at[oidx.at[j]], sem).wait()
                pltpu.async_copy(sel_k.at[pl.ds(j * 128, 128)], outk_hbm.at[oidx.at[j]], sem).wait()
                pltpu.async_copy(sel_i.at[pl.ds(j * 128, 128)], outi_hbm.at[oidx.at[j]], sem).wait()


def _sc_select(keys, xflat):
    mesh = plsc.VectorSubcoreMesh(core_axis_name="c", subcore_axis_name="s")
    f = functools.partial(
        pl.kernel,
        mesh=mesh,
        out_type=[
            jax.ShapeDtypeStruct((_OUTPAD,), jnp.int32),
            jax.ShapeDtypeStruct((_OUTPAD,), jnp.int32),
            jax.ShapeDtypeStruct((_OUTPAD,), jnp.float32),
        ],
        scratch_types=[
            pltpu.VMEM((_CHUNK,), jnp.int32),
            pltpu.VMEM((16 * 256,), jnp.int32),
            pltpu.VMEM((256,), jnp.int32),
            pltpu.VMEM((_NW * 256,), jnp.int32),
            pltpu.VMEM((_CHUNK + 16,), jnp.int32),
            pltpu.VMEM((_CHUNK + 16,), jnp.int32),
            pltpu.VMEM((_SELCAP + 16,), jnp.int32),
            pltpu.VMEM((_SELCAP + 16,), jnp.int32),
            pltpu.VMEM((_SELCAP + 16,), jnp.float32),
            pltpu.VMEM((16,), jnp.int32),
            pltpu.VMEM((_NW * 16,), jnp.int32),
            pltpu.VMEM((_NBATCH, 128), jnp.int32),
            pltpu.VMEM((_NBATCH, 128), jnp.int32),
            pltpu.VMEM_SHARED((_NW * 256,), jnp.int32),
            pltpu.VMEM_SHARED((_NW * 16,), jnp.int32),
            pltpu.SemaphoreType.DMA,
        ],
        compiler_params=pltpu.CompilerParams(
            needs_layout_passes=False, use_tc_tiling_on_sc=False
        ),
    )(_sc_body)
    return f(keys, xflat)


def _rank_body(k_ref, i_ref, x_ref, out_ref):
    keys = k_ref[...].reshape(1, _K)
    idx = i_ref[...].reshape(1, _K)
    x0 = x_ref[...].reshape(1, _K)
    kcol = keys.reshape(_K, 1)
    icol = idx.reshape(_K, 1)
    gt = (keys > kcol).astype(jnp.int32)
    tie = jnp.logical_and(keys == kcol, idx < icol).astype(jnp.int32)
    rank = jnp.sum(gt + tie, axis=1).reshape(1, _K)
    ku = lax.bitcast_convert_type(keys, jnp.uint32) ^ jnp.uint32(0x80000000)
    u = jnp.where((ku & jnp.uint32(0x80000000)) != 0, ku ^ jnp.uint32(0x80000000), ~ku)
    y = lax.bitcast_convert_type(u, jnp.float32)
    prod = (x0 * jax.nn.sigmoid(y)).reshape(_K, 1)
    row = lax.broadcasted_iota(jnp.int32, (_K, _K), 0)
    p2 = (row == rank).astype(jnp.float32)
    out = lax.dot_general(
        p2, prod, (((1,), (0,)), ((), ())),
        preferred_element_type=jnp.float32,
        precision=lax.Precision.HIGHEST,
    )
    out_ref[...] = out


def _tc_rank(ck, ci, cx):
    return pl.pallas_call(
        _rank_body,
        out_shape=jax.ShapeDtypeStruct((_K, 1), jnp.float32),
    )(ck, ci, cx)


def kernel(x, learnable_vector):
    keys = _tc_keys(x, learnable_vector)
    ck, ci, cx = _sc_select(keys, x.reshape(_N * _D))
    return _tc_rank(ck[:_K], ci[:_K], cx[:_K])
